# Initial kernel scaffold; baseline (speedup 1.0000x reference)
import jax
import jax.numpy as jnp
from jax.experimental import pallas as pl


def kernel(x, edge_index, W1, b1, W2, b2, W3, b3, fc1_w, fc1_b, fc2_w, fc2_b, fc3_w, fc3_b):
    raise NotImplementedError("write your pallas kernel here")


# speedup vs baseline: 18.9711x; 18.9711x over previous
import functools

import jax
import jax.numpy as jnp
from jax import lax
from jax.experimental import pallas as pl
from jax.experimental.pallas import tpu as pltpu
from jax.experimental.pallas import tpu_sc as plsc

N = 10000
NPAD = 10112
D_FEAT = 128
F = 64
E = 320000
NUM_CLASSES = 10

NC = 2
NS = 16
NW = NC * NS
K = 128
C = 80
EP = NW * C * K
ROWS_PER_TILE = NPAD // NS

_mesh = plsc.VectorSubcoreMesh(core_axis_name="c", subcore_axis_name="s",
                               num_cores=NC, num_subcores=NS)


def _zero_acc_slab(zbuf, acc, sid, width_rows):
    base = sid * ROWS_PER_TILE
    for off, n in ((0, 128), (128, 128), (256, 128), (384, 128), (512, 120)):
        pltpu.sync_copy(zbuf.at[pl.ds(0, n)], acc.at[pl.ds(base + off, n)])


def _deg_body(dst_hbm, out_hbm, acc, dst_v, ones_v, zbuf):
    cid = lax.axis_index("c")
    sid = lax.axis_index("s")
    wid = sid * NC + cid
    zero16 = jnp.zeros((16,), jnp.float32)
    one16 = jnp.ones((16,), jnp.float32)

    def fill(i, carry):
        zbuf[i, :] = zero16
        ones_v[i, :] = one16
        return carry

    lax.fori_loop(0, K, fill, 0)
    _zero_acc_slab(zbuf, acc, sid, 16)
    plsc.subcore_barrier()

    def body(g, carry):
        pltpu.sync_copy(dst_hbm.at[wid, g], dst_v.at[0])
        pltpu.sync_copy(ones_v, acc.at[dst_v.at[0]], add=True)
        return carry

    lax.fori_loop(0, C, body, 0)
    plsc.subcore_barrier()
    base = sid * ROWS_PER_TILE
    pltpu.sync_copy(acc.at[pl.ds(base, ROWS_PER_TILE)],
                    out_hbm.at[cid, pl.ds(base, ROWS_PER_TILE)])


NB = F // 16


def _scatter_body(u_hbm, src_hbm, dst_hbm, out_hbm,
                  acc, u_sh, src_v, dst_v, rows_v, zbuf, gsem):
    cid = lax.axis_index("c")
    sid = lax.axis_index("s")
    wid = sid * NC + cid
    zero16 = jnp.zeros((16,), jnp.float32)

    def fill(i, carry):
        for j in range(F // 16):
            zbuf[i, pl.ds(j * 16, 16)] = zero16
        return carry

    lax.fori_loop(0, K, fill, 0)
    base = sid * ROWS_PER_TILE
    pltpu.sync_copy(u_hbm.at[pl.ds(base, ROWS_PER_TILE)],
                    u_sh.at[pl.ds(base, ROWS_PER_TILE)])
    _zero_acc_slab(zbuf, acc, sid, F)
    plsc.subcore_barrier()

    def body(g, carry):
        pltpu.sync_copy(src_hbm.at[wid, g], src_v.at[0])
        pltpu.sync_copy(dst_hbm.at[wid, g], dst_v.at[0])
        pltpu.async_copy(u_sh.at[src_v.at[0]], rows_v.at[0], gsem).wait()
        pltpu.sync_copy(rows_v.at[0], acc.at[dst_v.at[0]], add=True)
        return carry

    lax.fori_loop(0, C, body, 0)
    plsc.subcore_barrier()
    for off, n in ((0, 128), (128, 128), (256, 128), (384, 128), (512, 120)):
        pltpu.sync_copy(acc.at[pl.ds(base + off, n)],
                        rows_v.at[0, pl.ds(0, n)])
        pltpu.sync_copy(rows_v.at[0, pl.ds(0, n)],
                        out_hbm.at[cid, pl.ds(base + off, n)])


def _make_deg_call(interpret=False):
    return pl.kernel(
        _deg_body,
        out_type=jax.ShapeDtypeStruct((NC, NPAD, 16), jnp.float32),
        mesh=_mesh,
        scratch_types=[
            pltpu.VMEM_SHARED((NPAD, 16), jnp.float32),
            pltpu.VMEM((1, K), jnp.int32),
            pltpu.VMEM((K, 16), jnp.float32),
            pltpu.VMEM((K, 16), jnp.float32),
        ],
        interpret=interpret,
    )


def _make_scatter_call(interpret=False):
    return pl.kernel(
        _scatter_body,
        out_type=jax.ShapeDtypeStruct((NC, NPAD, F), jnp.float32),
        mesh=_mesh,
        scratch_types=[
            pltpu.VMEM_SHARED((NPAD, F), jnp.float32),
            pltpu.VMEM_SHARED((NPAD, F), jnp.float32),
            pltpu.VMEM((1, K), jnp.int32),
            pltpu.VMEM((1, K), jnp.int32),
            pltpu.VMEM((1, K, F), jnp.float32),
            pltpu.VMEM((K, F), jnp.float32),
            pltpu.SemaphoreType.DMA,
        ],
        interpret=interpret,
    )


_deg_call = _make_deg_call()
_scatter_call = _make_scatter_call()


def _store_u(u_ref, u):
    u_ref[0:N, :] = u
    u_ref[N:NPAD, :] = jnp.zeros((NPAD - N, F), jnp.float32)


def _tc1_body(x_ref, w_ref, p_ref, u_ref, dis_ref):
    cnt = p_ref[0, 0:N, 0:1] + p_ref[1, 0:N, 0:1]
    dis = lax.rsqrt(cnt + 1.0)
    disf = jnp.broadcast_to(dis, (N, F))
    h = jnp.dot(x_ref[...], w_ref[...], preferred_element_type=jnp.float32)
    _store_u(u_ref, h * disf)
    dis_ref[...] = disf


def _tc_mid_body(a_ref, up_ref, dis_ref, w_ref, b_ref, u_ref, s_ref):
    disf = dis_ref[...]
    agg = a_ref[0, 0:N, :] + a_ref[1, 0:N, :] + up_ref[0:N, :]
    xl = jnp.maximum(agg * disf + b_ref[...], 0.0)
    s_ref[...] = jnp.sum(xl, axis=0, keepdims=True)
    _store_u(u_ref, jnp.dot(xl, w_ref[...],
                            preferred_element_type=jnp.float32) * disf)


def _tc_head_body(a_ref, up_ref, dis_ref, b3_ref, s1_ref, s2_ref,
                  fc1w_ref, fc1b_ref, fc2w_ref, fc2b_ref, fc3w_ref, fc3b_ref,
                  o_ref):
    agg = a_ref[0, 0:N, :] + a_ref[1, 0:N, :] + up_ref[0:N, :]
    x3 = jnp.maximum(agg * dis_ref[...] + b3_ref[...], 0.0)
    s3 = jnp.sum(x3, axis=0, keepdims=True)
    s = jnp.concatenate([s1_ref[...], s2_ref[...], s3], axis=1)
    h = jnp.maximum(jnp.dot(s, fc1w_ref[...],
                            preferred_element_type=jnp.float32) + fc1b_ref[...], 0.0)
    h = jnp.maximum(jnp.dot(h, fc2w_ref[...],
                            preferred_element_type=jnp.float32) + fc2b_ref[...], 0.0)
    o_ref[...] = jnp.dot(h, fc3w_ref[...],
                         preferred_element_type=jnp.float32) + fc3b_ref[...]


_tc1 = pl.pallas_call(
    _tc1_body,
    out_shape=[jax.ShapeDtypeStruct((NPAD, F), jnp.float32),
               jax.ShapeDtypeStruct((N, F), jnp.float32)],
)

_tc_mid = pl.pallas_call(
    _tc_mid_body,
    out_shape=[jax.ShapeDtypeStruct((NPAD, F), jnp.float32),
               jax.ShapeDtypeStruct((1, F), jnp.float32)],
)

_tc_head = pl.pallas_call(
    _tc_head_body,
    out_shape=jax.ShapeDtypeStruct((1, NUM_CLASSES), jnp.float32),
)


def kernel(x, edge_index, W1, b1, W2, b2, W3, b3,
           fc1_w, fc1_b, fc2_w, fc2_b, fc3_w, fc3_b):
    src = edge_index[0].astype(jnp.int32)
    dst = edge_index[1].astype(jnp.int32)
    pad_idx = N + (jnp.arange(EP - E, dtype=jnp.int32) % 16)
    srcp = jnp.concatenate([src, pad_idx]).reshape(NW, C, K)
    dstp = jnp.concatenate([dst, pad_idx]).reshape(NW, C, K)

    degp = _deg_call(dstp)
    u1, dis = _tc1(x, W1, degp)
    acc1 = _scatter_call(u1, srcp, dstp)
    u2, s1 = _tc_mid(acc1, u1, dis, W2, b1.reshape(1, F))
    acc2 = _scatter_call(u2, srcp, dstp)
    u3, s2 = _tc_mid(acc2, u2, dis, W3, b2.reshape(1, F))
    acc3 = _scatter_call(u3, srcp, dstp)
    out = _tc_head(acc3, u3, dis, b3.reshape(1, F), s1, s2,
                   fc1_w, fc1_b.reshape(1, -1), fc2_w, fc2_b.reshape(1, -1),
                   fc3_w, fc3_b.reshape(1, -1))
    return out.reshape(NUM_CLASSES)


# speedup vs baseline: 34.1283x; 1.7990x over previous
import functools

import jax
import jax.numpy as jnp
from jax import lax
from jax.experimental import pallas as pl
from jax.experimental.pallas import tpu as pltpu
from jax.experimental.pallas import tpu_sc as plsc

N = 10000
NPAD = 10112
D_FEAT = 128
F = 64
E = 320000
NUM_CLASSES = 10

NC = 2
NS = 16
NW = NC * NS
K = 128
C = 80
SUP = 8
NSUP = C // SUP
EP = NW * C * K
ROWS_PER_TILE = NPAD // NS

_mesh = plsc.VectorSubcoreMesh(core_axis_name="c", subcore_axis_name="s",
                               num_cores=NC, num_subcores=NS)


def _zero_acc_slab(zbuf, acc, sid, width_rows):
    base = sid * ROWS_PER_TILE
    for off, n in ((0, 128), (128, 128), (256, 128), (384, 128), (512, 120)):
        pltpu.sync_copy(zbuf.at[pl.ds(0, n)], acc.at[pl.ds(base + off, n)])


def _deg_body(dst_hbm, out_hbm, acc, dst_v, ones_v, zbuf, isem, ssem):
    cid = lax.axis_index("c")
    sid = lax.axis_index("s")
    wid = sid * NC + cid
    zero16 = jnp.zeros((16,), jnp.float32)
    one16 = jnp.ones((16,), jnp.float32)

    def fill(i, carry):
        zbuf[i, :] = zero16
        ones_v[i, :] = one16
        return carry

    lax.fori_loop(0, K, fill, 0)
    _zero_acc_slab(zbuf, acc, sid, 16)
    plsc.subcore_barrier()

    pltpu.sync_copy(dst_hbm.at[wid, pl.ds(0, SUP)], dst_v.at[0])

    def super_body(s, carry):
        b = lax.rem(s, 2)
        nb = 1 - b

        @pl.when(s + 1 < NSUP)
        def _():
            pltpu.async_copy(dst_hbm.at[wid, pl.ds((s + 1) * SUP, SUP)],
                             dst_v.at[nb], isem)

        for k in range(SUP):
            pltpu.async_copy(ones_v, acc.at[dst_v.at[b, k]], ssem, add=True)
        for k in range(SUP):
            pltpu.make_async_copy(ones_v, acc.at[dst_v.at[b, k]], ssem).wait()

        @pl.when(s + 1 < NSUP)
        def _():
            pltpu.make_async_copy(dst_hbm.at[wid, pl.ds((s + 1) * SUP, SUP)],
                                  dst_v.at[nb], isem).wait()
        return carry

    lax.fori_loop(0, NSUP, super_body, 0)
    plsc.subcore_barrier()
    base = sid * ROWS_PER_TILE
    pltpu.sync_copy(acc.at[pl.ds(base, ROWS_PER_TILE)],
                    out_hbm.at[cid, pl.ds(base, ROWS_PER_TILE)])


NB = F // 16


def _scatter_body(u_hbm, src_hbm, dst_hbm, out_hbm,
                  acc, u_sh, src_v, dst_v, rows_v, gsem, isem):
    cid = lax.axis_index("c")
    sid = lax.axis_index("s")
    wid = sid * NC + cid
    zero16 = jnp.zeros((16,), jnp.float32)

    def fill(i, carry):
        for j in range(F // 16):
            rows_v[0, i, pl.ds(j * 16, 16)] = zero16
        return carry

    lax.fori_loop(0, K, fill, 0)
    base = sid * ROWS_PER_TILE
    pltpu.sync_copy(u_hbm.at[pl.ds(base, ROWS_PER_TILE)],
                    u_sh.at[pl.ds(base, ROWS_PER_TILE)])
    for off, n in ((0, 128), (128, 128), (256, 128), (384, 128), (512, 120)):
        pltpu.sync_copy(rows_v.at[0, pl.ds(0, n)],
                        acc.at[pl.ds(base + off, n)])
    plsc.subcore_barrier()

    pltpu.sync_copy(src_hbm.at[wid, pl.ds(0, SUP)], src_v.at[0])
    pltpu.sync_copy(dst_hbm.at[wid, pl.ds(0, SUP)], dst_v.at[0])
    pltpu.async_copy(u_sh.at[src_v.at[0, 0]], rows_v.at[0], gsem)

    def super_body(s, carry):
        b = lax.rem(s, 2)
        nb = 1 - b

        @pl.when(s + 1 < NSUP)
        def _():
            pltpu.async_copy(src_hbm.at[wid, pl.ds((s + 1) * SUP, SUP)],
                             src_v.at[nb], isem)
            pltpu.async_copy(dst_hbm.at[wid, pl.ds((s + 1) * SUP, SUP)],
                             dst_v.at[nb], isem)

        for k in range(SUP):
            rb = k % 2
            pltpu.make_async_copy(u_sh.at[src_v.at[b, k]], rows_v.at[rb],
                                  gsem).wait()
            if k < SUP - 1:
                pltpu.async_copy(u_sh.at[src_v.at[b, k + 1]],
                                 rows_v.at[1 - rb], gsem)
            else:
                @pl.when(s + 1 < NSUP)
                def _():
                    pltpu.make_async_copy(
                        src_hbm.at[wid, pl.ds((s + 1) * SUP, SUP)],
                        src_v.at[nb], isem).wait()
                    pltpu.make_async_copy(
                        dst_hbm.at[wid, pl.ds((s + 1) * SUP, SUP)],
                        dst_v.at[nb], isem).wait()
                    pltpu.async_copy(u_sh.at[src_v.at[nb, 0]],
                                     rows_v.at[1 - rb], gsem)
            pltpu.sync_copy(rows_v.at[rb], acc.at[dst_v.at[b, k]], add=True)
        return carry

    lax.fori_loop(0, NSUP, super_body, 0)
    plsc.subcore_barrier()
    for off, n in ((0, 128), (128, 128), (256, 128), (384, 128), (512, 120)):
        pltpu.sync_copy(acc.at[pl.ds(base + off, n)],
                        rows_v.at[0, pl.ds(0, n)])
        pltpu.sync_copy(rows_v.at[0, pl.ds(0, n)],
                        out_hbm.at[cid, pl.ds(base + off, n)])


def _make_deg_call(interpret=False):
    return pl.kernel(
        _deg_body,
        out_type=jax.ShapeDtypeStruct((NC, NPAD, 16), jnp.float32),
        mesh=_mesh,
        scratch_types=[
            pltpu.VMEM_SHARED((NPAD, 16), jnp.float32),
            pltpu.VMEM((2, SUP, K), jnp.int32),
            pltpu.VMEM((K, 16), jnp.float32),
            pltpu.VMEM((K, 16), jnp.float32),
            pltpu.SemaphoreType.DMA,
            pltpu.SemaphoreType.DMA,
        ],
        interpret=interpret,
    )


def _make_scatter_call(interpret=False):
    return pl.kernel(
        _scatter_body,
        out_type=jax.ShapeDtypeStruct((NC, NPAD, F), jnp.float32),
        mesh=_mesh,
        scratch_types=[
            pltpu.VMEM_SHARED((NPAD, F), jnp.float32),
            pltpu.VMEM_SHARED((NPAD, F), jnp.float32),
            pltpu.VMEM((2, SUP, K), jnp.int32),
            pltpu.VMEM((2, SUP, K), jnp.int32),
            pltpu.VMEM((2, K, F), jnp.float32),
            pltpu.SemaphoreType.DMA,
            pltpu.SemaphoreType.DMA,
        ],
        interpret=interpret,
    )


_deg_call = _make_deg_call()
_scatter_call = _make_scatter_call()


def _store_u(u_ref, u):
    u_ref[0:N, :] = u
    u_ref[N:NPAD, :] = jnp.zeros((NPAD - N, F), jnp.float32)


def _tc1_body(x_ref, w_ref, p_ref, u_ref, dis_ref):
    cnt = p_ref[0, 0:N, 0:1] + p_ref[1, 0:N, 0:1]
    dis = lax.rsqrt(cnt + 1.0)
    disf = jnp.broadcast_to(dis, (N, F))
    h = jnp.dot(x_ref[...], w_ref[...], preferred_element_type=jnp.float32)
    _store_u(u_ref, h * disf)
    dis_ref[...] = disf


def _tc_mid_body(a_ref, up_ref, dis_ref, w_ref, b_ref, u_ref, s_ref):
    disf = dis_ref[...]
    agg = a_ref[0, 0:N, :] + a_ref[1, 0:N, :] + up_ref[0:N, :]
    xl = jnp.maximum(agg * disf + b_ref[...], 0.0)
    s_ref[...] = jnp.sum(xl, axis=0, keepdims=True)
    _store_u(u_ref, jnp.dot(xl, w_ref[...],
                            preferred_element_type=jnp.float32) * disf)


def _tc_head_body(a_ref, up_ref, dis_ref, b3_ref, s1_ref, s2_ref,
                  fc1w_ref, fc1b_ref, fc2w_ref, fc2b_ref, fc3w_ref, fc3b_ref,
                  o_ref):
    agg = a_ref[0, 0:N, :] + a_ref[1, 0:N, :] + up_ref[0:N, :]
    x3 = jnp.maximum(agg * dis_ref[...] + b3_ref[...], 0.0)
    s3 = jnp.sum(x3, axis=0, keepdims=True)
    s = jnp.concatenate([s1_ref[...], s2_ref[...], s3], axis=1)
    h = jnp.maximum(jnp.dot(s, fc1w_ref[...],
                            preferred_element_type=jnp.float32) + fc1b_ref[...], 0.0)
    h = jnp.maximum(jnp.dot(h, fc2w_ref[...],
                            preferred_element_type=jnp.float32) + fc2b_ref[...], 0.0)
    o_ref[...] = jnp.dot(h, fc3w_ref[...],
                         preferred_element_type=jnp.float32) + fc3b_ref[...]


_tc1 = pl.pallas_call(
    _tc1_body,
    out_shape=[jax.ShapeDtypeStruct((NPAD, F), jnp.float32),
               jax.ShapeDtypeStruct((N, F), jnp.float32)],
)

_tc_mid = pl.pallas_call(
    _tc_mid_body,
    out_shape=[jax.ShapeDtypeStruct((NPAD, F), jnp.float32),
               jax.ShapeDtypeStruct((1, F), jnp.float32)],
)

_tc_head = pl.pallas_call(
    _tc_head_body,
    out_shape=jax.ShapeDtypeStruct((1, NUM_CLASSES), jnp.float32),
)


def kernel(x, edge_index, W1, b1, W2, b2, W3, b3,
           fc1_w, fc1_b, fc2_w, fc2_b, fc3_w, fc3_b):
    src = edge_index[0].astype(jnp.int32)
    dst = edge_index[1].astype(jnp.int32)
    pad_idx = N + (jnp.arange(EP - E, dtype=jnp.int32) % 16)
    srcp = jnp.concatenate([src, pad_idx]).reshape(NW, C, K)
    dstp = jnp.concatenate([dst, pad_idx]).reshape(NW, C, K)

    degp = _deg_call(dstp)
    u1, dis = _tc1(x, W1, degp)
    acc1 = _scatter_call(u1, srcp, dstp)
    u2, s1 = _tc_mid(acc1, u1, dis, W2, b1.reshape(1, F))
    acc2 = _scatter_call(u2, srcp, dstp)
    u3, s2 = _tc_mid(acc2, u2, dis, W3, b2.reshape(1, F))
    acc3 = _scatter_call(u3, srcp, dstp)
    out = _tc_head(acc3, u3, dis, b3.reshape(1, F), s1, s2,
                   fc1_w, fc1_b.reshape(1, -1), fc2_w, fc2_b.reshape(1, -1),
                   fc3_w, fc3_b.reshape(1, -1))
    return out.reshape(NUM_CLASSES)
